# baseline (speedup 1.0000x reference)
import functools

import jax
import jax.numpy as jnp
from jax import lax
from jax.experimental import pallas as pl
from jax.experimental.pallas import tpu as pltpu
from jax.experimental.pallas import tpu_sc as plsc

B, L, H = 16, 2048, 1024
V = 50000
MAXPOS = 2048
P = 8

NC, NS = 2, 16
NW = NC * NS
LBLK = L // NW
C = 32
NSUB = LBLK // C
HV = H // 16


def _body(ids_hbm, vec_hbm, ipos_hbm, word_hbm, pemb_hbm, out_hbm,
          ids_v, ipos_v, vec_v, pos_v, out_v, gs0, gs1, ws0, ws1):
    cid = lax.axis_index("c")
    sid = lax.axis_index("s")
    wid = sid * NC + cid
    lb = wid * LBLK

    pltpu.sync_copy(ipos_hbm, ipos_v)
    pltpu.sync_copy(ids_hbm.at[pl.ds(wid * (B * LBLK), B * LBLK)], ids_v)
    pltpu.sync_copy(vec_hbm, vec_v)

    for s in range(NSUB):
        base = lb + s * C
        pltpu.sync_copy(pemb_hbm.at[pl.ds(base, C)], pos_v)

        def idx_ref(bb):
            return ids_v.at[pl.ds(bb * LBLK + s * C, C)]

        def issue_gather(bb, par):
            @pl.when(par == 0)
            def _():
                pltpu.async_copy(word_hbm.at[idx_ref(bb)], out_v.at[0], gs0)

            @pl.when(par == 1)
            def _():
                pltpu.async_copy(word_hbm.at[idx_ref(bb)], out_v.at[1], gs1)

        def wait_gather(par):
            @pl.when(par == 0)
            def _():
                pltpu.make_async_copy(
                    word_hbm.at[ids_v.at[pl.ds(0, C)]], out_v.at[0],
                    gs0).wait()

            @pl.when(par == 1)
            def _():
                pltpu.make_async_copy(
                    word_hbm.at[ids_v.at[pl.ds(0, C)]], out_v.at[1],
                    gs1).wait()

        def issue_wb(bb, par):
            @pl.when(par == 0)
            def _():
                pltpu.async_copy(out_v.at[0], out_hbm.at[bb, pl.ds(base, C)],
                                 ws0)

            @pl.when(par == 1)
            def _():
                pltpu.async_copy(out_v.at[1], out_hbm.at[bb, pl.ds(base, C)],
                                 ws1)

        def wait_wb(par):
            @pl.when(par == 0)
            def _():
                pltpu.make_async_copy(
                    out_v.at[0], out_hbm.at[0, pl.ds(base, C)], ws0).wait()

            @pl.when(par == 1)
            def _():
                pltpu.make_async_copy(
                    out_v.at[1], out_hbm.at[0, pl.ds(base, C)], ws1).wait()

        def seq_body(b, carry):
            par = b & 1
            wait_gather(par)

            @pl.when(b >= 1)
            def _():
                wait_wb(1 - par)

            @pl.when(b + 1 < B)
            def _():
                issue_gather(b + 1, 1 - par)

            def row_body(r, c2):
                for k in range(HV):
                    plsc.addupdate(out_v.at[par, r, pl.ds(k * 16, 16)],
                                   pos_v[r, pl.ds(k * 16, 16)])
                return c2
            lax.fori_loop(0, C, row_body, 0)

            rel = ipos_v[pl.ds(b * P, 16)] - base
            for j in range(P):
                pj = rel[j]

                @pl.when((pj >= 0) & (pj < C))
                def _():
                    def ins_k(k, c3):
                        out_v[par, pj, pl.ds(k * 16, 16)] = (
                            vec_v[pl.ds(b * H + k * 16, 16)]
                            + pos_v[pj, pl.ds(k * 16, 16)])
                        return c3
                    lax.fori_loop(0, HV, ins_k, 0)

            issue_wb(b, par)
            return carry

        issue_gather(0, 0)
        lax.fori_loop(0, B, seq_body, 0)
        pltpu.make_async_copy(
            out_v.at[1], out_hbm.at[0, pl.ds(base, C)], ws1).wait()


@jax.jit
def _run(ids_t, vec_flat, input_pos_flat, word_emb, pemb_shift):
    mesh = plsc.VectorSubcoreMesh(core_axis_name="c", subcore_axis_name="s",
                                  num_cores=NC, num_subcores=NS)
    f = pl.kernel(
        _body,
        out_type=jax.ShapeDtypeStruct((B, L, H), jnp.float32),
        mesh=mesh,
        scratch_types=[
            pltpu.VMEM((B * LBLK,), jnp.int32),
            pltpu.VMEM((B * P + 16,), jnp.int32),
            pltpu.VMEM((B * H,), jnp.float32),
            pltpu.VMEM((C, H), jnp.float32),
            pltpu.VMEM((2, C, H), jnp.float32),
            pltpu.SemaphoreType.DMA,
            pltpu.SemaphoreType.DMA,
            pltpu.SemaphoreType.DMA,
            pltpu.SemaphoreType.DMA,
        ],
    )
    return f(ids_t, vec_flat, input_pos_flat, word_emb, pemb_shift)


def kernel(input_ids, vectors, input_pos, word_emb, pos_emb):
    ids_t = (input_ids.astype(jnp.int32)
             .reshape(B, NW, LBLK).transpose(1, 0, 2).reshape(-1))
    ipos_flat = jnp.pad(input_pos.astype(jnp.int32).reshape(-1), (0, 16))
    pemb_shift = lax.slice(pos_emb, (1, 0), (L + 1, H))
    return _run(ids_t, vectors.reshape(-1), ipos_flat, word_emb, pemb_shift)
